# Initial kernel scaffold; baseline (speedup 1.0000x reference)
import functools

import jax
import jax.numpy as jnp
from jax import lax
from jax.experimental import pallas as pl
from jax.experimental.pallas import tpu as pltpu
from jax.experimental.pallas import tpu_sc as plsc

B = 1048576
W = 10
NUM_ROWS = 1024
L = 16

CH = 2048


def _encoder_body(bits_hbm, mat_hbm, out_hbm, tab_v, bits_v, out_v, nc, ns):
    nw = nc * ns
    rpt = B // nw
    nchunk = rpt // CH

    wid = lax.axis_index("s") * nc + lax.axis_index("c")
    row0 = wid * rpt

    lane = lax.iota(jnp.int32, L)
    zeros = jnp.zeros((L,), jnp.int32)
    ones = jnp.full((L,), 1, jnp.int32)

    pltpu.sync_copy(mat_hbm, tab_v)

    def _sq_body(j, acc):
        rows = j * L + lane
        c0 = plsc.load_gather(tab_v, [rows, zeros])
        c1 = plsc.load_gather(tab_v, [rows, ones])
        return acc + c0 * c0 + c1 * c1

    sq = lax.fori_loop(0, NUM_ROWS // L, _sq_body, jnp.zeros((L,), jnp.float32))
    mean_mag = jnp.sum(sq) * (1.0 / NUM_ROWS)
    mvec = jnp.full((L,), mean_mag)
    y = plsc.bitcast(jnp.full((L,), 0x5F3759DF, jnp.int32)
                     - (plsc.bitcast(mvec, jnp.int32) >> 1), jnp.float32)
    for _ in range(4):
        y = y * (1.5 - 0.5 * mvec * y * y)
    inv_nf = y

    def _inner(i, carry):
        rows = i * L + lane
        acc = jnp.zeros((L,), jnp.float32)
        for k in range(W):
            col = plsc.load_gather(bits_v, [rows, jnp.full((L,), k, jnp.int32)])
            acc = acc + col * float(2 ** (W - 1 - k))
        idx = acc.astype(jnp.int32)
        re = plsc.load_gather(tab_v, [idx, zeros]) * inv_nf
        im = plsc.load_gather(tab_v, [idx, ones]) * inv_nf
        plsc.store_scatter(out_v, [rows, zeros], re)
        plsc.store_scatter(out_v, [rows, ones], im)
        return carry

    def _chunk(c, carry):
        r0 = row0 + c * CH
        pltpu.sync_copy(bits_hbm.at[pl.ds(r0, CH)], bits_v)
        lax.fori_loop(0, CH // L, _inner, 0)
        pltpu.sync_copy(out_v, out_hbm.at[pl.ds(r0, CH)])
        return carry

    lax.fori_loop(0, nchunk, _chunk, 0)


def kernel(bit_sequence, matrix):
    mesh = plsc.VectorSubcoreMesh(core_axis_name="c", subcore_axis_name="s")
    nc, ns = mesh.num_cores, mesh.num_subcores
    body = functools.partial(_encoder_body, nc=nc, ns=ns)
    run = pl.kernel(
        body,
        out_type=jax.ShapeDtypeStruct((B, 2), jnp.float32),
        mesh=mesh,
        scratch_types=[
            pltpu.VMEM((NUM_ROWS, 2), jnp.float32),
            pltpu.VMEM((CH, W), jnp.float32),
            pltpu.VMEM((CH, 2), jnp.float32),
        ],
    )
    return run(bit_sequence, matrix)


# speedup vs baseline: 3.0062x; 3.0062x over previous
import functools

import jax
import jax.numpy as jnp
from jax import lax
from jax.experimental import pallas as pl
from jax.experimental.pallas import tpu as pltpu
from jax.experimental.pallas import tpu_sc as plsc

B = 1048576
W = 10
NUM_ROWS = 1024
L = 16

CH = 2048


def _encoder_body(bits_hbm, mat_hbm, out_hbm, tab_v, bits_v, out_v, nc, ns):
    nw = nc * ns
    rpt = B // nw
    nchunk = rpt // CH

    wid = lax.axis_index("s") * nc + lax.axis_index("c")
    row0 = wid * rpt

    lane = lax.iota(jnp.int32, L)
    zeros = jnp.zeros((L,), jnp.int32)
    ones = jnp.full((L,), 1, jnp.int32)

    pltpu.sync_copy(mat_hbm, tab_v)

    def _sq_body(j, acc):
        rows = j * L + lane
        c0 = plsc.load_gather(tab_v, [rows, zeros])
        c1 = plsc.load_gather(tab_v, [rows, ones])
        return acc + c0 * c0 + c1 * c1

    sq = lax.fori_loop(0, NUM_ROWS // L, _sq_body, jnp.zeros((L,), jnp.float32))
    mean_mag = jnp.sum(sq) * (1.0 / NUM_ROWS)
    mvec = jnp.full((L,), mean_mag)
    y = plsc.bitcast(jnp.full((L,), 0x5F3759DF, jnp.int32)
                     - (plsc.bitcast(mvec, jnp.int32) >> 1), jnp.float32)
    for _ in range(4):
        y = y * (1.5 - 0.5 * mvec * y * y)
    inv_nf = y

    def _inner(i, carry):
        rows = i * L + lane
        acc = jnp.zeros((L,), jnp.float32)
        for k in range(W):
            col = plsc.load_gather(bits_v, [rows, jnp.full((L,), k, jnp.int32)])
            acc = acc + col * float(2 ** (W - 1 - k))
        idx = acc.astype(jnp.int32)
        re = plsc.load_gather(tab_v, [idx, zeros]) * inv_nf
        im = plsc.load_gather(tab_v, [idx, ones]) * inv_nf
        plsc.store_scatter(out_v, [rows, zeros], re)
        plsc.store_scatter(out_v, [rows, ones], im)
        return carry

    def _chunk(c, carry):
        r0 = row0 + c * CH
        pltpu.sync_copy(bits_hbm.at[pl.ds(r0, CH)], bits_v)
        lax.fori_loop(0, CH // L, _inner, 0)
        pltpu.sync_copy(out_v, out_hbm.at[pl.ds(r0, CH)])
        return carry

    lax.fori_loop(0, nchunk, _chunk, 0)


def kernel(bit_sequence, matrix):
    mesh = plsc.VectorSubcoreMesh(core_axis_name="c", subcore_axis_name="s")
    nc, ns = mesh.num_cores, mesh.num_subcores
    body = functools.partial(_encoder_body, nc=nc, ns=ns)
    run = pl.kernel(
        body,
        out_type=jax.ShapeDtypeStruct((B, 2), jnp.float32),
        mesh=mesh,
        scratch_types=[
            pltpu.VMEM((NUM_ROWS, 2), jnp.float32),
            pltpu.VMEM((CH, W), jnp.float32),
            pltpu.VMEM((CH, 2), jnp.float32),
        ],
        compiler_params=pltpu.CompilerParams(
            needs_layout_passes=False, use_tc_tiling_on_sc=False),
    )
    return run(bit_sequence, matrix)


# speedup vs baseline: 3.2002x; 1.0645x over previous
import functools

import jax
import jax.numpy as jnp
from jax import lax
from jax.experimental import pallas as pl
from jax.experimental.pallas import tpu as pltpu
from jax.experimental.pallas import tpu_sc as plsc

B = 1048576
W = 10
NUM_ROWS = 1024
L = 16

CH = 2048
UNROLL = 8


def _encoder_body(bits_hbm, mat_hbm, out_hbm,
                  tab_v, bits_v0, bits_v1, out_v0, out_v1,
                  isem0, isem1, osem0, osem1, nc, ns):
    nw = nc * ns
    rpt = B // nw
    nchunk = rpt // CH
    npair = nchunk // 2

    wid = lax.axis_index("s") * nc + lax.axis_index("c")
    row0 = wid * rpt

    lane = lax.iota(jnp.int32, L)
    zeros = jnp.zeros((L,), jnp.int32)
    ones = jnp.full((L,), 1, jnp.int32)
    kcol = [jnp.full((L,), k, jnp.int32) for k in range(W)]

    def in_desc(c, buf, sem):
        return pltpu.make_async_copy(
            bits_hbm.at[pl.ds(row0 + c * CH, CH)], buf, sem)

    def out_desc(c, buf, sem):
        return pltpu.make_async_copy(
            buf, out_hbm.at[pl.ds(row0 + c * CH, CH)], sem)

    pltpu.sync_copy(mat_hbm, tab_v)

    def _sq_body(j, acc):
        rows = j * L + lane
        c0 = plsc.load_gather(tab_v, [rows, zeros])
        c1 = plsc.load_gather(tab_v, [rows, ones])
        return acc + c0 * c0 + c1 * c1

    sq = lax.fori_loop(0, NUM_ROWS // L, _sq_body, jnp.zeros((L,), jnp.float32))
    mean_mag = jnp.sum(sq) * (1.0 / NUM_ROWS)
    mvec = jnp.full((L,), mean_mag)
    y = plsc.bitcast(jnp.full((L,), 0x5F3759DF, jnp.int32)
                     - (plsc.bitcast(mvec, jnp.int32) >> 1), jnp.float32)
    for _ in range(4):
        y = y * (1.5 - 0.5 * mvec * y * y)
    inv_nf = y

    def compute_chunk(bits_v, out_v):
        @plsc.parallel_loop(0, CH // L, unroll=UNROLL)
        def _inner(i):
            rows = i * L + lane
            c = [plsc.load_gather(bits_v, [rows, kcol[k]]) for k in range(W)]
            s0 = c[0] * 512.0 + c[1] * 256.0
            s1 = c[2] * 128.0 + c[3] * 64.0
            s2 = c[4] * 32.0 + c[5] * 16.0
            s3 = c[6] * 8.0 + c[7] * 4.0
            s4 = c[8] * 2.0 + c[9]
            idx = ((s0 + s1) + (s2 + s3) + s4).astype(jnp.int32)
            re = plsc.load_gather(tab_v, [idx, zeros]) * inv_nf
            im = plsc.load_gather(tab_v, [idx, ones]) * inv_nf
            plsc.store_scatter(out_v, [rows, zeros], re)
            plsc.store_scatter(out_v, [rows, ones], im)

    in_desc(0, bits_v0, isem0).start()

    def pair_body(p, carry):
        c0 = 2 * p
        in_desc(c0 + 1, bits_v1, isem1).start()
        in_desc(c0, bits_v0, isem0).wait()

        @pl.when(p > 0)
        def _():
            out_desc(c0 - 2, out_v0, osem0).wait()
        compute_chunk(bits_v0, out_v0)
        out_desc(c0, out_v0, osem0).start()

        @pl.when(p + 1 < npair)
        def _():
            in_desc(c0 + 2, bits_v0, isem0).start()
        in_desc(c0 + 1, bits_v1, isem1).wait()

        @pl.when(p > 0)
        def _():
            out_desc(c0 - 1, out_v1, osem1).wait()
        compute_chunk(bits_v1, out_v1)
        out_desc(c0 + 1, out_v1, osem1).start()
        return carry

    lax.fori_loop(0, npair, pair_body, 0)
    out_desc(nchunk - 2, out_v0, osem0).wait()
    out_desc(nchunk - 1, out_v1, osem1).wait()


def kernel(bit_sequence, matrix):
    mesh = plsc.VectorSubcoreMesh(core_axis_name="c", subcore_axis_name="s")
    nc, ns = mesh.num_cores, mesh.num_subcores
    body = functools.partial(_encoder_body, nc=nc, ns=ns)
    run = pl.kernel(
        body,
        out_type=jax.ShapeDtypeStruct((B, 2), jnp.float32),
        mesh=mesh,
        scratch_types=[
            pltpu.VMEM((NUM_ROWS, 2), jnp.float32),
            pltpu.VMEM((CH, W), jnp.float32),
            pltpu.VMEM((CH, W), jnp.float32),
            pltpu.VMEM((CH, 2), jnp.float32),
            pltpu.VMEM((CH, 2), jnp.float32),
            pltpu.SemaphoreType.DMA,
            pltpu.SemaphoreType.DMA,
            pltpu.SemaphoreType.DMA,
            pltpu.SemaphoreType.DMA,
        ],
        compiler_params=pltpu.CompilerParams(
            needs_layout_passes=False, use_tc_tiling_on_sc=False),
    )
    return run(bit_sequence, matrix)


# speedup vs baseline: 4.4442x; 1.3887x over previous
import functools

import jax
import jax.numpy as jnp
from jax import lax
from jax.experimental import pallas as pl
from jax.experimental.pallas import tpu as pltpu
from jax.experimental.pallas import tpu_sc as plsc

B = 1048576
W = 10
NUM_ROWS = 1024
L = 16
PC = 128

CH = 128
UNROLL = 4


def _encoder_body(bits_hbm, mat_hbm, out_hbm,
                  tab_re, tab_im, piece_v,
                  bits_v0, bits_v1, out_v0, out_v1,
                  isem0, isem1, osem0, osem1, nc, ns):
    nw = nc * ns
    rpt = B // nw
    nchunk = rpt // CH
    npair = nchunk // 2

    wid = lax.axis_index("s") * nc + lax.axis_index("c")
    row0 = wid * rpt

    lane = lax.iota(jnp.int32, L)
    zeros = jnp.zeros((L,), jnp.int32)
    ones = jnp.full((L,), 1, jnp.int32)
    kcol = [jnp.full((L,), k, jnp.int32) for k in range(W)]

    def in_desc(c, buf, sem):
        return pltpu.make_async_copy(
            bits_hbm.at[pl.ds(row0 + c * CH, CH)], buf, sem)

    def out_desc(c, buf, sem):
        return pltpu.make_async_copy(
            buf, out_hbm.at[pl.ds(row0 + c * CH, CH)], sem)

    def _piece(j, carry):
        pltpu.sync_copy(mat_hbm.at[pl.ds(j * PC, PC)], piece_v)

        def _grp(g, carry2):
            rows = g * L + lane
            re = plsc.load_gather(piece_v, [rows, zeros])
            im = plsc.load_gather(piece_v, [rows, ones])
            tab_re[pl.ds(j * PC + g * L, L)] = re
            tab_im[pl.ds(j * PC + g * L, L)] = im
            return carry2

        lax.fori_loop(0, PC // L, _grp, 0)
        return carry

    lax.fori_loop(0, NUM_ROWS // PC, _piece, 0)

    def _sq_body(j, acc):
        re = tab_re[pl.ds(j * L, L)]
        im = tab_im[pl.ds(j * L, L)]
        return acc + re * re + im * im

    sq = lax.fori_loop(0, NUM_ROWS // L, _sq_body, jnp.zeros((L,), jnp.float32))
    mean_mag = jnp.sum(sq) * (1.0 / NUM_ROWS)
    mvec = jnp.full((L,), mean_mag)
    y = plsc.bitcast(jnp.full((L,), 0x5F3759DF, jnp.int32)
                     - (plsc.bitcast(mvec, jnp.int32) >> 1), jnp.float32)
    for _ in range(4):
        y = y * (1.5 - 0.5 * mvec * y * y)
    inv_nf = y

    def compute_chunk(bits_v, out_v):
        @plsc.parallel_loop(0, CH // L, unroll=UNROLL)
        def _inner(i):
            rows = i * L + lane
            c = [plsc.load_gather(bits_v, [rows, kcol[k]]) for k in range(W)]
            s0 = c[0] * 512.0 + c[1] * 256.0
            s1 = c[2] * 128.0 + c[3] * 64.0
            s2 = c[4] * 32.0 + c[5] * 16.0
            s3 = c[6] * 8.0 + c[7] * 4.0
            s4 = c[8] * 2.0 + c[9]
            idx = ((s0 + s1) + (s2 + s3) + s4).astype(jnp.int32)
            re = plsc.load_gather(tab_re, [idx]) * inv_nf
            im = plsc.load_gather(tab_im, [idx]) * inv_nf
            plsc.store_scatter(out_v, [rows, zeros], re)
            plsc.store_scatter(out_v, [rows, ones], im)

    in_desc(0, bits_v0, isem0).start()

    def pair_body(p, carry):
        c0 = 2 * p
        in_desc(c0 + 1, bits_v1, isem1).start()
        in_desc(c0, bits_v0, isem0).wait()

        @pl.when(p > 0)
        def _():
            out_desc(c0 - 2, out_v0, osem0).wait()
        compute_chunk(bits_v0, out_v0)
        out_desc(c0, out_v0, osem0).start()

        @pl.when(p + 1 < npair)
        def _():
            in_desc(c0 + 2, bits_v0, isem0).start()
        in_desc(c0 + 1, bits_v1, isem1).wait()

        @pl.when(p > 0)
        def _():
            out_desc(c0 - 1, out_v1, osem1).wait()
        compute_chunk(bits_v1, out_v1)
        out_desc(c0 + 1, out_v1, osem1).start()
        return carry

    lax.fori_loop(0, npair, pair_body, 0)
    out_desc(nchunk - 2, out_v0, osem0).wait()
    out_desc(nchunk - 1, out_v1, osem1).wait()


def kernel(bit_sequence, matrix):
    mesh = plsc.VectorSubcoreMesh(core_axis_name="c", subcore_axis_name="s")
    nc, ns = mesh.num_cores, mesh.num_subcores
    body = functools.partial(_encoder_body, nc=nc, ns=ns)
    run = pl.kernel(
        body,
        out_type=jax.ShapeDtypeStruct((B, 2), jnp.float32),
        mesh=mesh,
        scratch_types=[
            pltpu.VMEM((NUM_ROWS,), jnp.float32),
            pltpu.VMEM((NUM_ROWS,), jnp.float32),
            pltpu.VMEM((PC, 2), jnp.float32),
            pltpu.VMEM((CH, W), jnp.float32),
            pltpu.VMEM((CH, W), jnp.float32),
            pltpu.VMEM((CH, 2), jnp.float32),
            pltpu.VMEM((CH, 2), jnp.float32),
            pltpu.SemaphoreType.DMA,
            pltpu.SemaphoreType.DMA,
            pltpu.SemaphoreType.DMA,
            pltpu.SemaphoreType.DMA,
        ],
        compiler_params=pltpu.CompilerParams(
            needs_layout_passes=False, use_tc_tiling_on_sc=True),
    )
    return run(bit_sequence, matrix)
